# baseline (speedup 1.0000x reference)
import functools
import math

import jax
import jax.numpy as jnp
from jax import lax
from jax.experimental import pallas as pl
from jax.experimental.pallas import tpu as pltpu
from jax.experimental.pallas import tpu_sc as plsc

_NB = 8
_K = _NB + 1
_ALPHA = 0.5
_REG = 0.1
_N_OUTER = 3
_N_SINK = 5
_LOG_EPS = math.log(1e-30)

_BB = 256
_NPAD = 10240
_SC_NC = 2
_SC_NS = 16
_SC_CHUNK = 120


def _sc_mesh():
    return plsc.VectorSubcoreMesh(
        core_axis_name="c", subcore_axis_name="s",
        num_cores=_SC_NC, num_subcores=_SC_NS)


def _rank_in_vec(v):
    io = lax.iota(jnp.int32, 16)
    one = jnp.ones((16,), jnp.int32)
    zero = jnp.zeros((16,), jnp.int32)
    rank = zero
    nlater = zero
    for j in range(16):
        eq = jnp.where(v == v[j], one, zero)
        rank = rank + jnp.where(io > j, eq, zero)
        nlater = nlater + jnp.where(io < j, eq, zero)
    return rank, nlater


def _count_edges(src_pad, n_slots):
    ne_pad = src_pad.shape[0]
    nw = _SC_NC * _SC_NS
    chunk = ne_pad // nw

    @functools.partial(
        pl.kernel,
        out_type=jax.ShapeDtypeStruct((nw, n_slots // 128, 128), jnp.int32),
        mesh=_sc_mesh(),
        compiler_params=pltpu.CompilerParams(needs_layout_passes=False),
        scratch_types=[
            pltpu.VMEM((chunk,), jnp.int32),
            pltpu.VMEM((n_slots // 128, 128), jnp.int32),
        ],
    )
    def count_kernel(src_hbm, out_hbm, srcbuf, cnt):
        wid = lax.axis_index("s") * _SC_NC + lax.axis_index("c")
        zeros16 = jnp.zeros((16,), jnp.int32)

        def z_body(i, _):
            for j in range(8):
                cnt[i, pl.ds(j * 16, 16)] = zeros16
            return _

        lax.fori_loop(0, n_slots // 128, z_body, 0)
        pltpu.sync_copy(src_hbm.at[pl.ds(wid * chunk, chunk)], srcbuf)

        def c_body(i, _):
            v = srcbuf[pl.ds(i * 16, 16)]
            rank, nlater = _rank_in_vec(v)
            vh = v >> 7
            vl = v & 127
            c = plsc.load_gather(cnt, [vh, vl])
            plsc.store_scatter(cnt, [vh, vl], c + rank + nlater + 1,
                               mask=nlater == 0)
            return _

        lax.fori_loop(0, chunk // 16, c_body, 0)
        pltpu.sync_copy(cnt, out_hbm.at[wid])

    return count_kernel(src_pad)


def _assign_slots(src_pad, dst_pad, offs, n_slots):
    ne_pad = src_pad.shape[0]
    nw = _SC_NC * _SC_NS
    chunk = ne_pad // nw
    tbl_w = (_NB + 1) * n_slots

    @functools.partial(
        pl.kernel,
        out_type=jax.ShapeDtypeStruct((nw, tbl_w // 128, 128), jnp.int32),
        mesh=_sc_mesh(),
        compiler_params=pltpu.CompilerParams(needs_layout_passes=False),
        scratch_types=[
            pltpu.VMEM((chunk,), jnp.int32),
            pltpu.VMEM((chunk,), jnp.int32),
            pltpu.VMEM((n_slots // 128, 128), jnp.int32),
            pltpu.VMEM((tbl_w // 128, 128), jnp.int32),
        ],
    )
    def assign_kernel(src_hbm, dst_hbm, offs_hbm, out_hbm,
                      srcbuf, dstbuf, cnt, tbl):
        wid = lax.axis_index("s") * _SC_NC + lax.axis_index("c")
        zeros16 = jnp.zeros((16,), jnp.int32)

        def z_body(i, _):
            for j in range(8):
                tbl[i, pl.ds(j * 16, 16)] = zeros16
            return _

        lax.fori_loop(0, tbl_w // 128, z_body, 0)
        pltpu.sync_copy(src_hbm.at[pl.ds(wid * chunk, chunk)], srcbuf)
        pltpu.sync_copy(dst_hbm.at[pl.ds(wid * chunk, chunk)], dstbuf)
        pltpu.sync_copy(offs_hbm.at[wid], cnt)

        def a_body(i, _):
            v = srcbuf[pl.ds(i * 16, 16)]
            d = dstbuf[pl.ds(i * 16, 16)]
            rank, nlater = _rank_in_vec(v)
            vh = v >> 7
            vl = v & 127
            c = plsc.load_gather(cnt, [vh, vl])
            slot = c + rank
            r = jnp.minimum(slot, _NB)
            addr = r * n_slots + v
            plsc.store_scatter(tbl, [addr >> 7, addr & 127], d + 1)
            plsc.store_scatter(cnt, [vh, vl], slot + nlater + 1,
                               mask=nlater == 0)
            return _

        lax.fori_loop(0, chunk // 16, a_body, 0)
        pltpu.sync_copy(tbl, out_hbm.at[wid])

    return assign_kernel(src_pad, dst_pad, offs)


def _gather_rows(table, idx_flat):
    n_idx = idx_flat.shape[0]
    d = table.shape[1]
    nw = _SC_NC * _SC_NS
    per_w = n_idx // nw
    n_chunks = per_w // _SC_CHUNK
    assert per_w * nw == n_idx and n_chunks * _SC_CHUNK == per_w

    idx3 = idx_flat.reshape(nw, n_chunks, _SC_CHUNK)
    mesh = plsc.VectorSubcoreMesh(
        core_axis_name="c", subcore_axis_name="s",
        num_cores=_SC_NC, num_subcores=_SC_NS)

    @functools.partial(
        pl.kernel,
        out_type=jax.ShapeDtypeStruct((n_idx, d), jnp.float32),
        mesh=mesh,
        scratch_types=[
            pltpu.VMEM((n_chunks, _SC_CHUNK), jnp.int32),
            pltpu.VMEM((_SC_CHUNK, d), jnp.float32),
            pltpu.SemaphoreType.DMA,
        ],
    )
    def gather_kernel(table_hbm, idx_hbm, out_hbm, idx_v, rows_v, sem):
        wid = lax.axis_index("s") * _SC_NC + lax.axis_index("c")
        base = wid * per_w
        pltpu.sync_copy(idx_hbm.at[wid], idx_v)

        def body(i, _):
            pltpu.async_copy(table_hbm.at[idx_v.at[i]], rows_v, sem).wait()
            pltpu.sync_copy(rows_v, out_hbm.at[pl.ds(base + i * _SC_CHUNK,
                                                     _SC_CHUNK)])
            return _

        lax.fori_loop(0, n_chunks, body, 0)

    return gather_kernel(table, idx3)


def _solve_body(zg_ref, mk_ref, wm_ref, cqb_ref, c2b_ref, out_ref):
    k_ = zg_ref.shape[0]
    bb = zg_ref.shape[1]
    t_ = cqb_ref.shape[0]
    s_ = cqb_ref.shape[1]
    wm = wm_ref[...]
    cqb = cqb_ref[...]
    c2b = c2b_ref[...]

    ones8 = jnp.ones((bb, 8), jnp.float32)
    mh_ks = []
    for k in range(k_):
        f = zg_ref[k]
        fl = jnp.concatenate([f, f * f, ones8], axis=1)
        mt = jnp.dot(fl, wm, preferred_element_type=jnp.float32)
        mh_ks.append(mt.T.reshape(t_, 16, bb)[:, :s_, :])
    mh = jnp.stack(mh_ks, axis=0)

    mask = mk_ref[...]
    dm = jnp.sum(mask[1:], axis=0)
    deg = 1.0 + dm
    invdeg = 1.0 / deg
    logdeg = jnp.log(deg)
    logp = jnp.where(mask > 0.0, -logdeg[None, :], _LOG_EPS)
    logp4 = logp[:, None, None, :]
    logq = -math.log(float(s_))

    cp = jnp.concatenate([(dm * invdeg)[None], mask[1:] * invdeg[None, :]],
                         axis=0)
    cc = cp[:, None, None, :] + cqb[None]

    mask4 = mask[:, None, None, :]

    def tens_of(tp):
        a0 = jnp.sum(tp[1:] * mask4[1:], axis=0)
        ak = mask4[1:] * tp[0]
        a = jnp.concatenate([a0[None], ak], axis=0)
        ttm = jnp.zeros(a.shape, jnp.float32)
        for s in range(s_):
            ttm = ttm + a[:, :, s:s + 1, :] * c2b[None, :, s, :, :]
        return cc - 2.0 * ttm

    tp = jnp.broadcast_to((mask * (invdeg[None, :] / float(s_)))[:, None, None, :],
                          (k_, t_, s_, bb))

    for it in range(_N_OUTER):
        grad = mh + tens_of(tp)
        lk = grad * (-1.0 / _REG)
        ft = jnp.zeros((k_, t_, 1, bb), jnp.float32)
        gt = jnp.zeros((t_, s_, bb), jnp.float32)
        for _ in range(_N_SINK):
            xkg = lk + gt[None]
            mx = jnp.max(xkg, axis=2, keepdims=True)
            sm = jnp.sum(jnp.exp(xkg - mx), axis=2, keepdims=True)
            ft = logp4 - mx - jnp.log(sm)
            ykf = lk + ft
            mx2 = jnp.max(ykf, axis=0)
            sm2 = jnp.sum(jnp.exp(ykf - mx2[None]), axis=0)
            gt = logq - mx2 - jnp.log(sm2)
        gp = jnp.exp(lk + ft + gt[None])
        step = 2.0 / (it + 2.0)
        tp = (1.0 - step) * tp + step * gp

    val = (mh + 0.5 * tens_of(tp)) * tp
    out_ref[...] = jnp.sum(val, axis=(0, 2))


def _solve_call(zg, maskt, wm, cqb, c2b):
    k_, npad, d = zg.shape
    t_, s_ = cqb.shape[0], cqb.shape[1]
    grid = (npad // _BB,)
    return pl.pallas_call(
        _solve_body,
        grid=grid,
        in_specs=[
            pl.BlockSpec((k_, _BB, d), lambda i: (0, i, 0)),
            pl.BlockSpec((k_, _BB), lambda i: (0, i)),
            pl.BlockSpec(wm.shape, lambda i: (0, 0)),
            pl.BlockSpec(cqb.shape, lambda i: (0, 0, 0)),
            pl.BlockSpec(c2b.shape, lambda i: (0, 0, 0, 0)),
        ],
        out_specs=pl.BlockSpec((t_, _BB), lambda i: (0, i)),
        out_shape=jax.ShapeDtypeStruct((t_, npad), jnp.float32),
    )(zg, maskt, wm, cqb, c2b)


def kernel(x, edge_index, latent_template, templates_features):
    n, d = x.shape
    t_, s_, _ = latent_template.shape
    nb = _NB

    ne = edge_index.shape[1]
    nw = _SC_NC * _SC_NS
    ne_pad = ((ne + 16 * nw - 1) // (16 * nw)) * (16 * nw)
    src_pad = jnp.concatenate(
        [edge_index[0].astype(jnp.int32),
         jnp.full((ne_pad - ne,), _NPAD - 1, jnp.int32)])
    dst_pad = jnp.concatenate(
        [edge_index[1].astype(jnp.int32),
         jnp.zeros((ne_pad - ne,), jnp.int32)])
    cnts = _count_edges(src_pad, _NPAD).reshape(nw, _NPAD)
    offs = jnp.cumsum(cnts, axis=0) - cnts
    deg = jnp.sum(cnts, axis=0)
    tbls = _assign_slots(src_pad, dst_pad, offs.reshape(nw, _NPAD // 128, 128),
                         _NPAD)
    tbl = jnp.sum(tbls.reshape(nw, _K, _NPAD)[:, :nb, :], axis=0)
    self_row = jnp.arange(_NPAD, dtype=jnp.int32)
    nbf_t = jnp.where(tbl > 0, tbl - 1, self_row[None, :])
    idx_flat = jnp.concatenate([self_row[None, :], nbf_t], axis=0).reshape(-1)
    jj = jnp.arange(nb, dtype=jnp.int32)[:, None]
    maskt = jnp.concatenate(
        [(self_row < n)[None, :].astype(jnp.float32),
         (jj < deg[None, :]).astype(jnp.float32)], axis=0)

    c2 = 0.5 * (latent_template + jnp.transpose(latent_template, (0, 2, 1)))
    tf = templates_features
    y2 = jnp.sum(tf * tf, axis=-1)

    wtop = jnp.zeros((d, t_, 16), jnp.float32).at[:, :, :s_].set(
        -jnp.transpose(tf, (2, 0, 1)))
    wmid = jnp.zeros((d, t_, 16), jnp.float32).at[:, :, :s_].set(0.5)
    wbias = jnp.zeros((1, t_, 16), jnp.float32).at[0, :, :s_].set(0.5 * y2)
    wm = jnp.concatenate(
        [wtop.reshape(d, -1), wmid.reshape(d, -1), wbias.reshape(1, -1),
         jnp.zeros((7, t_ * 16), jnp.float32)], axis=0)

    cq = jnp.sum(c2 * c2, axis=2) / float(s_)
    cqb = jnp.broadcast_to(cq[:, :, None], (t_, s_, _BB)).astype(jnp.float32)
    c2b = jnp.broadcast_to(c2[:, :, :, None], (t_, s_, s_, _BB)).astype(jnp.float32)

    zg = _gather_rows(x, idx_flat).reshape(_K, _NPAD, d)

    out = _solve_call(zg, maskt, wm, cqb, c2b)
    return out[:, :n].T


# speedup vs baseline: 1.0647x; 1.0647x over previous
import functools
import math

import jax
import jax.numpy as jnp
from jax import lax
from jax.experimental import pallas as pl
from jax.experimental.pallas import tpu as pltpu
from jax.experimental.pallas import tpu_sc as plsc

_NB = 8
_K = _NB + 1
_ALPHA = 0.5
_REG = 0.1
_N_OUTER = 3
_N_SINK = 5
_LOG_EPS = math.log(1e-30)

_BB = 128
_NPAD = 10240
_SC_NC = 2
_SC_NS = 16
_SC_CHUNK = 120


def _sc_mesh():
    return plsc.VectorSubcoreMesh(
        core_axis_name="c", subcore_axis_name="s",
        num_cores=_SC_NC, num_subcores=_SC_NS)


def _rank_in_vec(v):
    io = lax.iota(jnp.int32, 16)
    one = jnp.ones((16,), jnp.int32)
    zero = jnp.zeros((16,), jnp.int32)
    rank = zero
    nlater = zero
    for j in range(16):
        eq = jnp.where(v == v[j], one, zero)
        rank = rank + jnp.where(io > j, eq, zero)
        nlater = nlater + jnp.where(io < j, eq, zero)
    return rank, nlater


def _count_edges(src_pad, n_slots):
    ne_pad = src_pad.shape[0]
    nw = _SC_NC * _SC_NS
    chunk = ne_pad // nw

    @functools.partial(
        pl.kernel,
        out_type=jax.ShapeDtypeStruct((nw, n_slots // 128, 128), jnp.int32),
        mesh=_sc_mesh(),
        compiler_params=pltpu.CompilerParams(needs_layout_passes=False),
        scratch_types=[
            pltpu.VMEM((chunk,), jnp.int32),
            pltpu.VMEM((n_slots // 128, 128), jnp.int32),
        ],
    )
    def count_kernel(src_hbm, out_hbm, srcbuf, cnt):
        wid = lax.axis_index("s") * _SC_NC + lax.axis_index("c")
        zeros16 = jnp.zeros((16,), jnp.int32)

        def z_body(i, _):
            for j in range(8):
                cnt[i, pl.ds(j * 16, 16)] = zeros16
            return _

        lax.fori_loop(0, n_slots // 128, z_body, 0)
        pltpu.sync_copy(src_hbm.at[pl.ds(wid * chunk, chunk)], srcbuf)

        def c_body(i, _):
            v = srcbuf[pl.ds(i * 16, 16)]
            rank, nlater = _rank_in_vec(v)
            vh = v >> 7
            vl = v & 127
            c = plsc.load_gather(cnt, [vh, vl])
            plsc.store_scatter(cnt, [vh, vl], c + rank + nlater + 1,
                               mask=nlater == 0)
            return _

        lax.fori_loop(0, chunk // 16, c_body, 0)
        pltpu.sync_copy(cnt, out_hbm.at[wid])

    return count_kernel(src_pad)


def _assign_slots(src_pad, dst_pad, offs, n_slots):
    ne_pad = src_pad.shape[0]
    nw = _SC_NC * _SC_NS
    chunk = ne_pad // nw
    tbl_w = (_NB + 1) * n_slots

    @functools.partial(
        pl.kernel,
        out_type=jax.ShapeDtypeStruct((nw, tbl_w // 128, 128), jnp.int32),
        mesh=_sc_mesh(),
        compiler_params=pltpu.CompilerParams(needs_layout_passes=False),
        scratch_types=[
            pltpu.VMEM((chunk,), jnp.int32),
            pltpu.VMEM((chunk,), jnp.int32),
            pltpu.VMEM((n_slots // 128, 128), jnp.int32),
            pltpu.VMEM((tbl_w // 128, 128), jnp.int32),
        ],
    )
    def assign_kernel(src_hbm, dst_hbm, offs_hbm, out_hbm,
                      srcbuf, dstbuf, cnt, tbl):
        wid = lax.axis_index("s") * _SC_NC + lax.axis_index("c")
        zeros16 = jnp.zeros((16,), jnp.int32)

        def z_body(i, _):
            for j in range(8):
                tbl[i, pl.ds(j * 16, 16)] = zeros16
            return _

        lax.fori_loop(0, tbl_w // 128, z_body, 0)
        pltpu.sync_copy(src_hbm.at[pl.ds(wid * chunk, chunk)], srcbuf)
        pltpu.sync_copy(dst_hbm.at[pl.ds(wid * chunk, chunk)], dstbuf)
        pltpu.sync_copy(offs_hbm.at[wid], cnt)

        def a_body(i, _):
            v = srcbuf[pl.ds(i * 16, 16)]
            d = dstbuf[pl.ds(i * 16, 16)]
            rank, nlater = _rank_in_vec(v)
            vh = v >> 7
            vl = v & 127
            c = plsc.load_gather(cnt, [vh, vl])
            slot = c + rank
            r = jnp.minimum(slot, _NB)
            addr = r * n_slots + v
            plsc.store_scatter(tbl, [addr >> 7, addr & 127], d + 1)
            plsc.store_scatter(cnt, [vh, vl], slot + nlater + 1,
                               mask=nlater == 0)
            return _

        lax.fori_loop(0, chunk // 16, a_body, 0)
        pltpu.sync_copy(tbl, out_hbm.at[wid])

    return assign_kernel(src_pad, dst_pad, offs)


def _gather_rows(table, idx_flat):
    n_idx = idx_flat.shape[0]
    d = table.shape[1]
    nw = _SC_NC * _SC_NS
    per_w = n_idx // nw
    n_chunks = per_w // _SC_CHUNK
    assert per_w * nw == n_idx and n_chunks * _SC_CHUNK == per_w

    idx3 = idx_flat.reshape(nw, n_chunks, _SC_CHUNK)
    mesh = plsc.VectorSubcoreMesh(
        core_axis_name="c", subcore_axis_name="s",
        num_cores=_SC_NC, num_subcores=_SC_NS)

    @functools.partial(
        pl.kernel,
        out_type=jax.ShapeDtypeStruct((n_idx, d), jnp.float32),
        mesh=mesh,
        scratch_types=[
            pltpu.VMEM((n_chunks, _SC_CHUNK), jnp.int32),
            pltpu.VMEM((_SC_CHUNK, d), jnp.float32),
            pltpu.VMEM((_SC_CHUNK, d), jnp.float32),
            pltpu.SemaphoreType.DMA,
            pltpu.SemaphoreType.DMA,
        ],
    )
    def gather_kernel(table_hbm, idx_hbm, out_hbm, idx_v, rows0, rows1,
                      sem0, sem1):
        wid = lax.axis_index("s") * _SC_NC + lax.axis_index("c")
        base = wid * per_w
        pltpu.sync_copy(idx_hbm.at[wid], idx_v)
        cp0 = pltpu.async_copy(table_hbm.at[idx_v.at[0]], rows0, sem0)

        def body(i, _):
            i1 = jnp.minimum(2 * i + 1, n_chunks - 1)
            pltpu.async_copy(table_hbm.at[idx_v.at[i1]], rows1, sem1)
            pltpu.make_async_copy(table_hbm.at[idx_v.at[0]], rows0,
                                  sem0).wait()
            pltpu.sync_copy(rows0, out_hbm.at[pl.ds(base + 2 * i * _SC_CHUNK,
                                                    _SC_CHUNK)])
            i2 = jnp.minimum(2 * i + 2, n_chunks - 1)
            pltpu.async_copy(table_hbm.at[idx_v.at[i2]], rows0, sem0)
            pltpu.make_async_copy(table_hbm.at[idx_v.at[0]], rows1,
                                  sem1).wait()
            pltpu.sync_copy(rows1,
                            out_hbm.at[pl.ds(base + (2 * i + 1) * _SC_CHUNK,
                                             _SC_CHUNK)])
            return _

        lax.fori_loop(0, n_chunks // 2, body, 0)
        pltpu.make_async_copy(table_hbm.at[idx_v.at[0]], rows0, sem0).wait()

    return gather_kernel(table, idx3)


def _solve_body(zg_ref, mk_ref, wm_ref, cqb_ref, c2b_ref, out_ref):
    k_ = zg_ref.shape[0]
    bb = zg_ref.shape[1]
    t_ = cqb_ref.shape[0]
    s_ = cqb_ref.shape[1]
    wm = wm_ref[...]
    cqb = cqb_ref[...]
    c2b = c2b_ref[...]

    ones8 = jnp.ones((bb, 8), jnp.float32)
    mh_ks = []
    for k in range(k_):
        f = zg_ref[k]
        fl = jnp.concatenate([f, f * f, ones8], axis=1)
        mt = jnp.dot(fl, wm, preferred_element_type=jnp.float32)
        mh_ks.append(mt.T.reshape(t_, 16, bb)[:, :s_, :])
    mh = jnp.stack(mh_ks, axis=0)

    mask = mk_ref[...]
    dm = jnp.sum(mask[1:], axis=0)
    deg = 1.0 + dm
    invdeg = 1.0 / deg
    logdeg = jnp.log(deg)
    logp = jnp.where(mask > 0.0, -logdeg[None, :], _LOG_EPS)
    logp4 = logp[:, None, None, :]
    logq = -math.log(float(s_))

    cp = jnp.concatenate([(dm * invdeg)[None], mask[1:] * invdeg[None, :]],
                         axis=0)
    cc = cp[:, None, None, :] + cqb[None]

    mask4 = mask[:, None, None, :]

    def tens_of(tp):
        a0 = jnp.sum(tp[1:] * mask4[1:], axis=0)
        ak = mask4[1:] * tp[0]
        a = jnp.concatenate([a0[None], ak], axis=0)
        ttm = jnp.zeros(a.shape, jnp.float32)
        for s in range(s_):
            ttm = ttm + a[:, :, s:s + 1, :] * c2b[None, :, s, :, :]
        return cc - 2.0 * ttm

    tp = jnp.broadcast_to((mask * (invdeg[None, :] / float(s_)))[:, None, None, :],
                          (k_, t_, s_, bb))

    for it in range(_N_OUTER):
        grad = mh + tens_of(tp)
        lk = grad * (-1.0 / _REG)
        ft = jnp.zeros((k_, t_, 1, bb), jnp.float32)
        gt = jnp.zeros((t_, s_, bb), jnp.float32)
        for _ in range(_N_SINK):
            xkg = lk + gt[None]
            mx = jnp.max(xkg, axis=2, keepdims=True)
            sm = jnp.sum(jnp.exp(xkg - mx), axis=2, keepdims=True)
            ft = logp4 - mx - jnp.log(sm)
            ykf = lk + ft
            mx2 = jnp.max(ykf, axis=0)
            sm2 = jnp.sum(jnp.exp(ykf - mx2[None]), axis=0)
            gt = logq - mx2 - jnp.log(sm2)
        gp = jnp.exp(lk + ft + gt[None])
        step = 2.0 / (it + 2.0)
        tp = (1.0 - step) * tp + step * gp

    val = (mh + 0.5 * tens_of(tp)) * tp
    out_ref[...] = jnp.sum(val, axis=(0, 2))


def _solve_call(zg, maskt, wm, cqb, c2b):
    k_, npad, d = zg.shape
    t_, s_ = cqb.shape[0], cqb.shape[1]
    grid = (npad // _BB,)
    return pl.pallas_call(
        _solve_body,
        grid=grid,
        in_specs=[
            pl.BlockSpec((k_, _BB, d), lambda i: (0, i, 0)),
            pl.BlockSpec((k_, _BB), lambda i: (0, i)),
            pl.BlockSpec(wm.shape, lambda i: (0, 0)),
            pl.BlockSpec(cqb.shape, lambda i: (0, 0, 0)),
            pl.BlockSpec(c2b.shape, lambda i: (0, 0, 0, 0)),
        ],
        out_specs=pl.BlockSpec((t_, _BB), lambda i: (0, i)),
        out_shape=jax.ShapeDtypeStruct((t_, npad), jnp.float32),
    )(zg, maskt, wm, cqb, c2b)


def kernel(x, edge_index, latent_template, templates_features):
    n, d = x.shape
    t_, s_, _ = latent_template.shape
    nb = _NB

    ne = edge_index.shape[1]
    nw = _SC_NC * _SC_NS
    ne_pad = ((ne + 16 * nw - 1) // (16 * nw)) * (16 * nw)
    src_pad = jnp.concatenate(
        [edge_index[0].astype(jnp.int32),
         jnp.full((ne_pad - ne,), _NPAD - 1, jnp.int32)])
    dst_pad = jnp.concatenate(
        [edge_index[1].astype(jnp.int32),
         jnp.zeros((ne_pad - ne,), jnp.int32)])
    cnts = _count_edges(src_pad, _NPAD).reshape(nw, _NPAD)
    offs = jnp.cumsum(cnts, axis=0) - cnts
    deg = jnp.sum(cnts, axis=0)
    tbls = _assign_slots(src_pad, dst_pad, offs.reshape(nw, _NPAD // 128, 128),
                         _NPAD)
    tbl = jnp.sum(tbls.reshape(nw, _K, _NPAD)[:, :nb, :], axis=0)
    self_row = jnp.arange(_NPAD, dtype=jnp.int32)
    nbf_t = jnp.where(tbl > 0, tbl - 1, self_row[None, :])
    idx_flat = jnp.concatenate([self_row[None, :], nbf_t], axis=0).reshape(-1)
    jj = jnp.arange(nb, dtype=jnp.int32)[:, None]
    maskt = jnp.concatenate(
        [(self_row < n)[None, :].astype(jnp.float32),
         (jj < deg[None, :]).astype(jnp.float32)], axis=0)

    c2 = 0.5 * (latent_template + jnp.transpose(latent_template, (0, 2, 1)))
    tf = templates_features
    y2 = jnp.sum(tf * tf, axis=-1)

    wtop = jnp.zeros((d, t_, 16), jnp.float32).at[:, :, :s_].set(
        -jnp.transpose(tf, (2, 0, 1)))
    wmid = jnp.zeros((d, t_, 16), jnp.float32).at[:, :, :s_].set(0.5)
    wbias = jnp.zeros((1, t_, 16), jnp.float32).at[0, :, :s_].set(0.5 * y2)
    wm = jnp.concatenate(
        [wtop.reshape(d, -1), wmid.reshape(d, -1), wbias.reshape(1, -1),
         jnp.zeros((7, t_ * 16), jnp.float32)], axis=0)

    cq = jnp.sum(c2 * c2, axis=2) / float(s_)
    cqb = jnp.broadcast_to(cq[:, :, None], (t_, s_, _BB)).astype(jnp.float32)
    c2b = jnp.broadcast_to(c2[:, :, :, None], (t_, s_, s_, _BB)).astype(jnp.float32)

    zg = _gather_rows(x, idx_flat).reshape(_K, _NPAD, d)

    out = _solve_call(zg, maskt, wm, cqb, c2b)
    return out[:, :n].T
